# baseline (speedup 1.0000x reference)
import functools

import jax
import jax.numpy as jnp
from jax import lax
from jax.experimental import pallas as pl
from jax.experimental.pallas import tpu as pltpu
from jax.experimental.pallas import tpu_sc as plsc

N = 10000
E = 320000
D = 128
G = 64
L = 5
C = 2

RB = 1000
NBLK = N // RB

NC = 2
NS = 16
NW = NC * NS
EPW = E // NW
EB = 40
NCHUNK = EPW // EB
NBUF = 3
NIB = 4
RPT = 624
REM = N - RPT * NS
ZB = 104


def _sc_message_body(hin, eidx, ea, out, acc, idxbuf, rows, eav,
                     isem, gsem, esem, ssem):
    c = lax.axis_index("c")
    s = lax.axis_index("s")
    wid = c * NS + s
    ebase = wid * EPW

    def issue_idx(ci):
        ib = lax.rem(ci, NIB)
        pltpu.async_copy(eidx.at[wid, ci], idxbuf.at[ib], isem.at[ib])

    def wait_idx(ci):
        ib = lax.rem(ci, NIB)
        pltpu.make_async_copy(eidx.at[wid, ci], idxbuf.at[ib],
                              isem.at[ib]).wait()

    def issue_in(ci, b):
        ib = lax.rem(ci, NIB)
        pltpu.async_copy(hin.at[idxbuf.at[ib, 0]], rows.at[b], gsem.at[b])
        pltpu.async_copy(ea.at[pl.ds(ebase + ci * EB, EB)],
                         eav.at[b], esem.at[b])

    def wait_in(ci, b):
        ib = lax.rem(ci, NIB)
        pltpu.make_async_copy(hin.at[idxbuf.at[ib, 0]], rows.at[b],
                              gsem.at[b]).wait()
        pltpu.make_async_copy(ea.at[pl.ds(ebase + ci * EB, EB)],
                              eav.at[b], esem.at[b]).wait()

    def issue_scatter(ci, b):
        ib = lax.rem(ci, NIB)
        pltpu.async_copy(rows.at[b], acc.at[idxbuf.at[ib, 1]], ssem.at[b],
                         add=True)

    def wait_scatter(ci, b):
        ib = lax.rem(ci, NIB)
        pltpu.make_async_copy(rows.at[b], acc.at[idxbuf.at[ib, 1]],
                              ssem.at[b]).wait()

    def compute(b):
        def rowop(r, rcarry):
            for kk in range(8):
                sl = pl.ds(kk * 16, 16)
                rows[b, r, sl] = jnp.maximum(rows[b, r, sl] + eav[b, r, sl],
                                             0.0)
            return rcarry

        lax.fori_loop(0, EB, rowop, 0)

    issue_idx(0)
    issue_idx(1)
    issue_idx(2)
    wait_idx(0)
    wait_idx(1)
    issue_in(0, 0)
    issue_in(1, 1)

    zero = jnp.zeros((16,), jnp.float32)

    def zfill(i, carry):
        r = i // 8
        k = (i % 8) * 16
        eav[2, r, pl.ds(k, 16)] = zero
        return carry

    lax.fori_loop(0, EB * 8, zfill, 0)
    for j in range(RPT // EB):
        pltpu.sync_copy(eav.at[2], acc.at[pl.ds(s * RPT + j * EB, EB)])
    pltpu.sync_copy(eav.at[2].at[pl.ds(0, RPT - (RPT // EB) * EB)],
                    acc.at[pl.ds(s * RPT + (RPT // EB) * EB,
                                 RPT - (RPT // EB) * EB)])

    @pl.when(s == 0)
    def _():
        pltpu.sync_copy(eav.at[2].at[pl.ds(0, REM)],
                        acc.at[pl.ds(RPT * NS, REM)])

    plsc.subcore_barrier()

    def step(ci, carry):
        b = lax.rem(ci, NBUF)
        bn = lax.rem(ci + 2, NBUF)
        wait_in(ci, b)
        compute(b)
        issue_scatter(ci, b)

        @pl.when(ci + 2 < NCHUNK)
        def _():
            @pl.when(ci >= 1)
            def _():
                wait_scatter(ci - 1, bn)

            wait_idx(ci + 2)
            issue_in(ci + 2, bn)

        @pl.when(ci + 3 < NCHUNK)
        def _():
            issue_idx(ci + 3)

        return carry

    lax.fori_loop(0, NCHUNK, step, 0)

    for t in range(NBUF):
        ci = NCHUNK - NBUF + t
        wait_scatter(ci, ci % NBUF)
    plsc.subcore_barrier()

    pltpu.sync_copy(acc.at[pl.ds(s * RPT, RPT)],
                    out.at[c, pl.ds(s * RPT, RPT)])

    @pl.when(s == 0)
    def _():
        pltpu.sync_copy(acc.at[pl.ds(RPT * NS, REM)],
                        out.at[c, pl.ds(RPT * NS, REM)])


@jax.jit
def _sc_message(hin, eidx, ea):
    mesh = plsc.VectorSubcoreMesh(core_axis_name="c", subcore_axis_name="s",
                                  num_cores=NC, num_subcores=NS)
    return pl.kernel(
        _sc_message_body,
        out_type=jax.ShapeDtypeStruct((NC, N, D), jnp.float32),
        mesh=mesh,
        scratch_types=[
            pltpu.VMEM_SHARED((N, D), jnp.float32),
            pltpu.VMEM((NIB, 2, EB), jnp.int32),
            pltpu.VMEM((NBUF, EB, D), jnp.float32),
            pltpu.VMEM((NBUF, EB, D), jnp.float32),
            pltpu.SemaphoreType.DMA((NIB,)),
            pltpu.SemaphoreType.DMA((NBUF,)),
            pltpu.SemaphoreType.DMA((NBUF,)),
            pltpu.SemaphoreType.DMA((NBUF,)),
        ],
    )(hin, eidx, ea)


def _tc_pre_body(h_ref, vn_ref, b_ref, hin_ref, seg_ref):
    i = pl.program_id(0)
    bk = b_ref[0]
    onehot = (bk.T == lax.broadcasted_iota(jnp.int32, (RB, G), 1)
              ).astype(jnp.float32)
    hin = h_ref[...] + jnp.dot(onehot, vn_ref[...],
                               preferred_element_type=jnp.float32)
    hin_ref[...] = hin
    contrib = jnp.dot(onehot.T, hin, preferred_element_type=jnp.float32)

    @pl.when(i == 0)
    def _():
        seg_ref[...] = contrib

    @pl.when(i != 0)
    def _():
        seg_ref[...] += contrib


@jax.jit
def _tc_pre(h, vn, batch3):
    return pl.pallas_call(
        _tc_pre_body,
        grid=(NBLK,),
        in_specs=[
            pl.BlockSpec((RB, D), lambda i: (i, 0)),
            pl.BlockSpec((G, D), lambda i: (0, 0)),
            pl.BlockSpec((1, 1, RB), lambda i: (i, 0, 0)),
        ],
        out_specs=[
            pl.BlockSpec((RB, D), lambda i: (i, 0)),
            pl.BlockSpec((G, D), lambda i: (0, 0)),
        ],
        out_shape=[
            jax.ShapeDtypeStruct((N, D), jnp.float32),
            jax.ShapeDtypeStruct((G, D), jnp.float32),
        ],
    )(h, vn, batch3)


def _tc_mlp_body(hin_ref, agg_ref, eps_ref, w1_ref, a1_ref, c1_ref, w2_ref,
                 a2_ref, c2_ref, out_ref, *, final_relu):
    z = (1.0 + eps_ref[0, 0]) * hin_ref[...] + agg_ref[0] + agg_ref[1]
    z1 = jnp.dot(z, w1_ref[...].T, preferred_element_type=jnp.float32)
    z1 = z1 * a1_ref[...] + c1_ref[...]
    z1 = jnp.maximum(z1, 0.0)
    z2 = jnp.dot(z1, w2_ref[...].T, preferred_element_type=jnp.float32)
    z2 = z2 * a2_ref[...] + c2_ref[...]
    if final_relu:
        z2 = jnp.maximum(z2, 0.0)
    out_ref[...] = z2


@functools.partial(jax.jit, static_argnames=("final_relu",))
def _tc_mlp(hin, agg2, eps, w1, a1, c1, w2, a2, c2, *, final_relu):
    body = functools.partial(_tc_mlp_body, final_relu=final_relu)
    return pl.pallas_call(
        body,
        grid=(NBLK,),
        in_specs=[
            pl.BlockSpec((RB, D), lambda i: (i, 0)),
            pl.BlockSpec((NC, RB, D), lambda i: (0, i, 0)),
            pl.BlockSpec((1, 1), lambda i: (0, 0)),
            pl.BlockSpec((2 * D, D), lambda i: (0, 0)),
            pl.BlockSpec((1, 2 * D), lambda i: (0, 0)),
            pl.BlockSpec((1, 2 * D), lambda i: (0, 0)),
            pl.BlockSpec((D, 2 * D), lambda i: (0, 0)),
            pl.BlockSpec((1, D), lambda i: (0, 0)),
            pl.BlockSpec((1, D), lambda i: (0, 0)),
        ],
        out_specs=pl.BlockSpec((RB, D), lambda i: (i, 0)),
        out_shape=jax.ShapeDtypeStruct((N, D), jnp.float32),
    )(hin, agg2, eps, w1, a1, c1, w2, a2, c2)


def _tc_vn_body(seg_ref, vn_ref, w1_ref, a1_ref, c1_ref, w2_ref, a2_ref,
                c2_ref, out_ref):
    vt = seg_ref[...] + vn_ref[...]
    v = jnp.dot(vt, w1_ref[...].T, preferred_element_type=jnp.float32)
    v = v * a1_ref[...] + c1_ref[...]
    v = jnp.maximum(v, 0.0)
    v = jnp.dot(v, w2_ref[...].T, preferred_element_type=jnp.float32)
    v = v * a2_ref[...] + c2_ref[...]
    out_ref[...] = jnp.maximum(v, 0.0)


@jax.jit
def _tc_vn(seg, vn, w1, a1, c1, w2, a2, c2):
    return pl.pallas_call(
        _tc_vn_body,
        out_shape=jax.ShapeDtypeStruct((G, D), jnp.float32),
    )(seg, vn, w1, a1, c1, w2, a2, c2)


def _tc_pool_body(h_ref, b_ref, wh_ref, bh_ref, out_ref, sum_ref, cnt_ref):
    i = pl.program_id(0)
    bk = b_ref[0]
    onehot = (bk.T == lax.broadcasted_iota(jnp.int32, (RB, G), 1)
              ).astype(jnp.float32)

    @pl.when(i == 0)
    def _():
        sum_ref[...] = jnp.zeros((G, D), jnp.float32)
        cnt_ref[...] = jnp.zeros((G, D), jnp.float32)

    sum_ref[...] += jnp.dot(onehot.T, h_ref[...],
                            preferred_element_type=jnp.float32)
    cnt_ref[...] += jnp.dot(onehot.T, jnp.ones((RB, D), jnp.float32),
                            preferred_element_type=jnp.float32)

    @pl.when(i == NBLK - 1)
    def _():
        hg = sum_ref[...] / jnp.maximum(cnt_ref[...], 1.0)
        out_ref[...] = jnp.dot(hg, wh_ref[...].T,
                               preferred_element_type=jnp.float32) + bh_ref[...]


@jax.jit
def _tc_pool(h, batch3, wh_pad, bh_pad):
    return pl.pallas_call(
        _tc_pool_body,
        grid=(NBLK,),
        in_specs=[
            pl.BlockSpec((RB, D), lambda i: (i, 0)),
            pl.BlockSpec((1, 1, RB), lambda i: (i, 0, 0)),
            pl.BlockSpec((D, D), lambda i: (0, 0)),
            pl.BlockSpec((1, D), lambda i: (0, 0)),
        ],
        out_specs=pl.BlockSpec((G, D), lambda i: (0, 0)),
        out_shape=jax.ShapeDtypeStruct((G, D), jnp.float32),
        scratch_shapes=[
            pltpu.VMEM((G, D), jnp.float32),
            pltpu.VMEM((G, D), jnp.float32),
        ],
    )(h, batch3, wh_pad, bh_pad)


def kernel(x, edge_index, edge_attr, batch, params):
    eidx = jnp.stack([edge_index[0].reshape(NW, NCHUNK, EB),
                      edge_index[1].reshape(NW, NCHUNK, EB)], axis=2)
    batch3 = batch.reshape(NBLK, 1, RB)

    p = params
    wh_pad = jnp.zeros((D, D), jnp.float32).at[:C, :].set(p['Wh'])
    bh_pad = jnp.zeros((1, D), jnp.float32).at[0, :C].set(p['bh'])

    h = x
    vn = jnp.zeros((G, D), jnp.float32)
    for l in range(L):
        hin, seg = _tc_pre(h, vn, batch3)
        agg2 = _sc_message(hin, eidx, edge_attr)
        h = _tc_mlp(
            hin, agg2,
            p['eps'][l].reshape(1, 1),
            p['W1'][l], p['bn1_g'][l].reshape(1, 2 * D),
            (p['b1'][l] * p['bn1_g'][l] + p['bn1_b'][l]).reshape(1, 2 * D),
            p['W2'][l], p['bn_g'][l].reshape(1, D),
            (p['b2'][l] * p['bn_g'][l] + p['bn_b'][l]).reshape(1, D),
            final_relu=(l < L - 1),
        )
        if l < L - 1:
            vn = _tc_vn(
                seg, vn,
                p['vW1'][l], p['vbn1_g'][l].reshape(1, 2 * D),
                (p['vb1'][l] * p['vbn1_g'][l] + p['vbn1_b'][l]).reshape(1, 2 * D),
                p['vW2'][l], p['vbn2_g'][l].reshape(1, D),
                (p['vb2'][l] * p['vbn2_g'][l] + p['vbn2_b'][l]).reshape(1, D),
            )
    out = _tc_pool(h, batch3, wh_pad, bh_pad)
    return out[:, :C]


# speedup vs baseline: 2.2987x; 2.2987x over previous
import functools

import jax
import jax.numpy as jnp
from jax import lax
from jax.experimental import pallas as pl
from jax.experimental.pallas import tpu as pltpu
from jax.experimental.pallas import tpu_sc as plsc

N = 10000
E = 320000
D = 128
G = 64
L = 5
C = 2

RB = 1000
NBLK = N // RB

NC = 2
NS = 16
NW = NC * NS
EPW = E // NW
EB = 40
NCHUNK = EPW // EB
NBUF = 3
NIB = 4
RPT = 624
REM = N - RPT * NS
ZB = 104


def _sc_message_body(hin, eidx, ea, out, acc, idxbuf, rows, eav,
                     isem, gsem, esem, ssem):
    c = lax.axis_index("c")
    s = lax.axis_index("s")
    wid = c * NS + s
    ebase = wid * EPW

    def issue_idx(ci):
        ib = lax.rem(ci, NIB)
        pltpu.async_copy(eidx.at[wid, ci], idxbuf.at[ib], isem.at[ib])

    def wait_idx(ci):
        ib = lax.rem(ci, NIB)
        pltpu.make_async_copy(eidx.at[wid, ci], idxbuf.at[ib],
                              isem.at[ib]).wait()

    def issue_in(ci, b):
        ib = lax.rem(ci, NIB)
        pltpu.async_copy(hin.at[idxbuf.at[ib, 0]], rows.at[b], gsem.at[b])
        pltpu.async_copy(ea.at[pl.ds(ebase + ci * EB, EB)],
                         eav.at[b], esem.at[b])

    def wait_in(ci, b):
        ib = lax.rem(ci, NIB)
        pltpu.make_async_copy(hin.at[idxbuf.at[ib, 0]], rows.at[b],
                              gsem.at[b]).wait()
        pltpu.make_async_copy(ea.at[pl.ds(ebase + ci * EB, EB)],
                              eav.at[b], esem.at[b]).wait()

    def issue_scatter(ci, b):
        ib = lax.rem(ci, NIB)
        pltpu.async_copy(rows.at[b], acc.at[idxbuf.at[ib, 1]], ssem.at[b],
                         add=True)

    def wait_scatter(ci, b):
        ib = lax.rem(ci, NIB)
        pltpu.make_async_copy(rows.at[b], acc.at[idxbuf.at[ib, 1]],
                              ssem.at[b]).wait()

    def compute(b):
        @plsc.parallel_loop(0, EB, unroll=4)
        def _(r):
            for kk in range(8):
                sl = pl.ds(kk * 16, 16)
                rows[b, r, sl] = jnp.maximum(rows[b, r, sl] + eav[b, r, sl],
                                             0.0)

    issue_idx(0)
    issue_idx(1)
    issue_idx(2)
    wait_idx(0)
    wait_idx(1)
    issue_in(0, 0)
    issue_in(1, 1)

    zero = jnp.zeros((16,), jnp.float32)

    def zfill(i, carry):
        r = i // 8
        k = (i % 8) * 16
        eav[2, r, pl.ds(k, 16)] = zero
        return carry

    lax.fori_loop(0, EB * 8, zfill, 0)
    for j in range(RPT // EB):
        pltpu.sync_copy(eav.at[2], acc.at[pl.ds(s * RPT + j * EB, EB)])
    pltpu.sync_copy(eav.at[2].at[pl.ds(0, RPT - (RPT // EB) * EB)],
                    acc.at[pl.ds(s * RPT + (RPT // EB) * EB,
                                 RPT - (RPT // EB) * EB)])

    @pl.when(s == 0)
    def _():
        pltpu.sync_copy(eav.at[2].at[pl.ds(0, REM)],
                        acc.at[pl.ds(RPT * NS, REM)])

    plsc.subcore_barrier()

    def stepfn(ci, b):
        bn = (b + 2) % NBUF
        wait_in(ci, b)
        compute(b)
        issue_scatter(ci, b)

        @pl.when(ci + 2 < NCHUNK)
        def _():
            @pl.when(ci >= 1)
            def _():
                wait_scatter(ci - 1, bn)

            wait_idx(ci + 2)
            issue_in(ci + 2, bn)

        @pl.when(ci + 3 < NCHUNK)
        def _():
            issue_idx(ci + 3)

    def group(g, carry):
        for b in range(NBUF):
            stepfn(NBUF * g + b, b)
        return carry

    lax.fori_loop(0, NCHUNK // NBUF, group, 0)
    stepfn(NCHUNK - 1, (NCHUNK - 1) % NBUF)

    for t in range(NBUF):
        ci = NCHUNK - NBUF + t
        wait_scatter(ci, ci % NBUF)
    plsc.subcore_barrier()

    pltpu.sync_copy(acc.at[pl.ds(s * RPT, RPT)],
                    out.at[c, pl.ds(s * RPT, RPT)])

    @pl.when(s == 0)
    def _():
        pltpu.sync_copy(acc.at[pl.ds(RPT * NS, REM)],
                        out.at[c, pl.ds(RPT * NS, REM)])


@jax.jit
def _sc_message(hin, eidx, ea):
    mesh = plsc.VectorSubcoreMesh(core_axis_name="c", subcore_axis_name="s",
                                  num_cores=NC, num_subcores=NS)
    return pl.kernel(
        _sc_message_body,
        out_type=jax.ShapeDtypeStruct((NC, N, D), jnp.float32),
        mesh=mesh,
        scratch_types=[
            pltpu.VMEM_SHARED((N, D), jnp.float32),
            pltpu.VMEM((NIB, 2, EB), jnp.int32),
            pltpu.VMEM((NBUF, EB, D), jnp.float32),
            pltpu.VMEM((NBUF, EB, D), jnp.float32),
            pltpu.SemaphoreType.DMA((NIB,)),
            pltpu.SemaphoreType.DMA((NBUF,)),
            pltpu.SemaphoreType.DMA((NBUF,)),
            pltpu.SemaphoreType.DMA((NBUF,)),
        ],
    )(hin, eidx, ea)


def _tc_pre_body(h_ref, vn_ref, b_ref, hin_ref, seg_ref):
    i = pl.program_id(0)
    bk = b_ref[0]
    onehot = (bk.T == lax.broadcasted_iota(jnp.int32, (RB, G), 1)
              ).astype(jnp.float32)
    hin = h_ref[...] + jnp.dot(onehot, vn_ref[...],
                               preferred_element_type=jnp.float32)
    hin_ref[...] = hin
    contrib = jnp.dot(onehot.T, hin, preferred_element_type=jnp.float32)

    @pl.when(i == 0)
    def _():
        seg_ref[...] = contrib

    @pl.when(i != 0)
    def _():
        seg_ref[...] += contrib


@jax.jit
def _tc_pre(h, vn, batch3):
    return pl.pallas_call(
        _tc_pre_body,
        grid=(NBLK,),
        in_specs=[
            pl.BlockSpec((RB, D), lambda i: (i, 0)),
            pl.BlockSpec((G, D), lambda i: (0, 0)),
            pl.BlockSpec((1, 1, RB), lambda i: (i, 0, 0)),
        ],
        out_specs=[
            pl.BlockSpec((RB, D), lambda i: (i, 0)),
            pl.BlockSpec((G, D), lambda i: (0, 0)),
        ],
        out_shape=[
            jax.ShapeDtypeStruct((N, D), jnp.float32),
            jax.ShapeDtypeStruct((G, D), jnp.float32),
        ],
    )(h, vn, batch3)


def _tc_mlp_body(hin_ref, agg_ref, eps_ref, w1_ref, a1_ref, c1_ref, w2_ref,
                 a2_ref, c2_ref, out_ref, *, final_relu):
    z = (1.0 + eps_ref[0, 0]) * hin_ref[...] + agg_ref[0] + agg_ref[1]
    z1 = jnp.dot(z, w1_ref[...].T, preferred_element_type=jnp.float32)
    z1 = z1 * a1_ref[...] + c1_ref[...]
    z1 = jnp.maximum(z1, 0.0)
    z2 = jnp.dot(z1, w2_ref[...].T, preferred_element_type=jnp.float32)
    z2 = z2 * a2_ref[...] + c2_ref[...]
    if final_relu:
        z2 = jnp.maximum(z2, 0.0)
    out_ref[...] = z2


@functools.partial(jax.jit, static_argnames=("final_relu",))
def _tc_mlp(hin, agg2, eps, w1, a1, c1, w2, a2, c2, *, final_relu):
    body = functools.partial(_tc_mlp_body, final_relu=final_relu)
    return pl.pallas_call(
        body,
        grid=(NBLK,),
        in_specs=[
            pl.BlockSpec((RB, D), lambda i: (i, 0)),
            pl.BlockSpec((NC, RB, D), lambda i: (0, i, 0)),
            pl.BlockSpec((1, 1), lambda i: (0, 0)),
            pl.BlockSpec((2 * D, D), lambda i: (0, 0)),
            pl.BlockSpec((1, 2 * D), lambda i: (0, 0)),
            pl.BlockSpec((1, 2 * D), lambda i: (0, 0)),
            pl.BlockSpec((D, 2 * D), lambda i: (0, 0)),
            pl.BlockSpec((1, D), lambda i: (0, 0)),
            pl.BlockSpec((1, D), lambda i: (0, 0)),
        ],
        out_specs=pl.BlockSpec((RB, D), lambda i: (i, 0)),
        out_shape=jax.ShapeDtypeStruct((N, D), jnp.float32),
    )(hin, agg2, eps, w1, a1, c1, w2, a2, c2)


def _tc_vn_body(seg_ref, vn_ref, w1_ref, a1_ref, c1_ref, w2_ref, a2_ref,
                c2_ref, out_ref):
    vt = seg_ref[...] + vn_ref[...]
    v = jnp.dot(vt, w1_ref[...].T, preferred_element_type=jnp.float32)
    v = v * a1_ref[...] + c1_ref[...]
    v = jnp.maximum(v, 0.0)
    v = jnp.dot(v, w2_ref[...].T, preferred_element_type=jnp.float32)
    v = v * a2_ref[...] + c2_ref[...]
    out_ref[...] = jnp.maximum(v, 0.0)


@jax.jit
def _tc_vn(seg, vn, w1, a1, c1, w2, a2, c2):
    return pl.pallas_call(
        _tc_vn_body,
        out_shape=jax.ShapeDtypeStruct((G, D), jnp.float32),
    )(seg, vn, w1, a1, c1, w2, a2, c2)


def _tc_pool_body(h_ref, b_ref, wh_ref, bh_ref, out_ref, sum_ref, cnt_ref):
    i = pl.program_id(0)
    bk = b_ref[0]
    onehot = (bk.T == lax.broadcasted_iota(jnp.int32, (RB, G), 1)
              ).astype(jnp.float32)

    @pl.when(i == 0)
    def _():
        sum_ref[...] = jnp.zeros((G, D), jnp.float32)
        cnt_ref[...] = jnp.zeros((G, D), jnp.float32)

    sum_ref[...] += jnp.dot(onehot.T, h_ref[...],
                            preferred_element_type=jnp.float32)
    cnt_ref[...] += jnp.dot(onehot.T, jnp.ones((RB, D), jnp.float32),
                            preferred_element_type=jnp.float32)

    @pl.when(i == NBLK - 1)
    def _():
        hg = sum_ref[...] / jnp.maximum(cnt_ref[...], 1.0)
        out_ref[...] = jnp.dot(hg, wh_ref[...].T,
                               preferred_element_type=jnp.float32) + bh_ref[...]


@jax.jit
def _tc_pool(h, batch3, wh_pad, bh_pad):
    return pl.pallas_call(
        _tc_pool_body,
        grid=(NBLK,),
        in_specs=[
            pl.BlockSpec((RB, D), lambda i: (i, 0)),
            pl.BlockSpec((1, 1, RB), lambda i: (i, 0, 0)),
            pl.BlockSpec((D, D), lambda i: (0, 0)),
            pl.BlockSpec((1, D), lambda i: (0, 0)),
        ],
        out_specs=pl.BlockSpec((G, D), lambda i: (0, 0)),
        out_shape=jax.ShapeDtypeStruct((G, D), jnp.float32),
        scratch_shapes=[
            pltpu.VMEM((G, D), jnp.float32),
            pltpu.VMEM((G, D), jnp.float32),
        ],
    )(h, batch3, wh_pad, bh_pad)


def kernel(x, edge_index, edge_attr, batch, params):
    eidx = jnp.stack([edge_index[0].reshape(NW, NCHUNK, EB),
                      edge_index[1].reshape(NW, NCHUNK, EB)], axis=2)
    batch3 = batch.reshape(NBLK, 1, RB)

    p = params
    wh_pad = jnp.zeros((D, D), jnp.float32).at[:C, :].set(p['Wh'])
    bh_pad = jnp.zeros((1, D), jnp.float32).at[0, :C].set(p['bh'])

    h = x
    vn = jnp.zeros((G, D), jnp.float32)
    for l in range(L):
        hin, seg = _tc_pre(h, vn, batch3)
        agg2 = _sc_message(hin, eidx, edge_attr)
        h = _tc_mlp(
            hin, agg2,
            p['eps'][l].reshape(1, 1),
            p['W1'][l], p['bn1_g'][l].reshape(1, 2 * D),
            (p['b1'][l] * p['bn1_g'][l] + p['bn1_b'][l]).reshape(1, 2 * D),
            p['W2'][l], p['bn_g'][l].reshape(1, D),
            (p['b2'][l] * p['bn_g'][l] + p['bn_b'][l]).reshape(1, D),
            final_relu=(l < L - 1),
        )
        if l < L - 1:
            vn = _tc_vn(
                seg, vn,
                p['vW1'][l], p['vbn1_g'][l].reshape(1, 2 * D),
                (p['vb1'][l] * p['vbn1_g'][l] + p['vbn1_b'][l]).reshape(1, 2 * D),
                p['vW2'][l], p['vbn2_g'][l].reshape(1, D),
                (p['vb2'][l] * p['vbn2_g'][l] + p['vbn2_b'][l]).reshape(1, D),
            )
    out = _tc_pool(h, batch3, wh_pad, bh_pad)
    return out[:, :C]


# speedup vs baseline: 2.3945x; 1.0417x over previous
import functools

import jax
import jax.numpy as jnp
from jax import lax
from jax.experimental import pallas as pl
from jax.experimental.pallas import tpu as pltpu
from jax.experimental.pallas import tpu_sc as plsc

N = 10000
E = 320000
D = 128
G = 64
L = 5
C = 2

RB = 1000
NBLK = N // RB

NC = 2
NS = 16
NW = NC * NS
EPW = E // NW
EB = 40
NCHUNK = EPW // EB
NBUF = 3
NIB = 4
RPT = 624
REM = N - RPT * NS
ZB = 104


def _sc_message_body(hin, eidx, ea, out, acc, idxbuf, rows, eav,
                     isem, gsem, esem, ssem):
    c = lax.axis_index("c")
    s = lax.axis_index("s")
    wid = c * NS + s
    ebase = wid * EPW

    def issue_idx(ci):
        ib = lax.rem(ci, NIB)
        pltpu.async_copy(eidx.at[wid, ci], idxbuf.at[ib], isem.at[ib])

    def wait_idx(ci):
        ib = lax.rem(ci, NIB)
        pltpu.make_async_copy(eidx.at[wid, ci], idxbuf.at[ib],
                              isem.at[ib]).wait()

    def issue_in(ci, b):
        ib = lax.rem(ci, NIB)
        pltpu.async_copy(hin.at[idxbuf.at[ib, 0]], rows.at[b], gsem.at[b])
        pltpu.async_copy(ea.at[pl.ds(ebase + ci * EB, EB)],
                         eav.at[b], esem.at[b])

    def wait_in(ci, b):
        ib = lax.rem(ci, NIB)
        pltpu.make_async_copy(hin.at[idxbuf.at[ib, 0]], rows.at[b],
                              gsem.at[b]).wait()
        pltpu.make_async_copy(ea.at[pl.ds(ebase + ci * EB, EB)],
                              eav.at[b], esem.at[b]).wait()

    def issue_scatter(ci, b):
        ib = lax.rem(ci, NIB)
        pltpu.async_copy(rows.at[b], acc.at[idxbuf.at[ib, 1]], ssem.at[b],
                         add=True)

    def wait_scatter(ci, b):
        ib = lax.rem(ci, NIB)
        pltpu.make_async_copy(rows.at[b], acc.at[idxbuf.at[ib, 1]],
                              ssem.at[b]).wait()

    def compute(b):
        @plsc.parallel_loop(0, EB, unroll=4)
        def _(r):
            for kk in range(8):
                sl = pl.ds(kk * 16, 16)
                rows[b, r, sl] = jnp.maximum(rows[b, r, sl] + eav[b, r, sl],
                                             0.0)

    issue_idx(0)
    issue_idx(1)
    issue_idx(2)
    wait_idx(0)
    wait_idx(1)
    issue_in(0, 0)
    issue_in(1, 1)

    zero = jnp.zeros((16,), jnp.float32)

    def zfill(i, carry):
        r = i // 8
        k = (i % 8) * 16
        eav[2, r, pl.ds(k, 16)] = zero
        return carry

    lax.fori_loop(0, EB * 8, zfill, 0)
    for j in range(RPT // EB):
        pltpu.sync_copy(eav.at[2], acc.at[pl.ds(s * RPT + j * EB, EB)])
    pltpu.sync_copy(eav.at[2].at[pl.ds(0, RPT - (RPT // EB) * EB)],
                    acc.at[pl.ds(s * RPT + (RPT // EB) * EB,
                                 RPT - (RPT // EB) * EB)])

    @pl.when(s == 0)
    def _():
        pltpu.sync_copy(eav.at[2].at[pl.ds(0, REM)],
                        acc.at[pl.ds(RPT * NS, REM)])

    plsc.subcore_barrier()

    def stepfn(ci, b):
        bn = (b + 2) % NBUF
        wait_in(ci, b)
        compute(b)
        issue_scatter(ci, b)

        @pl.when(ci + 2 < NCHUNK)
        def _():
            @pl.when(ci >= 1)
            def _():
                wait_scatter(ci - 1, bn)

            wait_idx(ci + 2)
            issue_in(ci + 2, bn)

        @pl.when(ci + 3 < NCHUNK)
        def _():
            issue_idx(ci + 3)

    def group(g, carry):
        for b in range(NBUF):
            stepfn(NBUF * g + b, b)
        return carry

    lax.fori_loop(0, NCHUNK // NBUF, group, 0)
    stepfn(NCHUNK - 1, (NCHUNK - 1) % NBUF)

    for t in range(NBUF):
        ci = NCHUNK - NBUF + t
        wait_scatter(ci, ci % NBUF)
    plsc.subcore_barrier()

    pltpu.sync_copy(acc.at[pl.ds(s * RPT, RPT)],
                    out.at[c, pl.ds(s * RPT, RPT)])

    @pl.when(s == 0)
    def _():
        pltpu.sync_copy(acc.at[pl.ds(RPT * NS, REM)],
                        out.at[c, pl.ds(RPT * NS, REM)])


@jax.jit
def _sc_message(hin, eidx, ea):
    mesh = plsc.VectorSubcoreMesh(core_axis_name="c", subcore_axis_name="s",
                                  num_cores=NC, num_subcores=NS)
    return pl.kernel(
        _sc_message_body,
        out_type=jax.ShapeDtypeStruct((NC, N, D), jnp.float32),
        mesh=mesh,
        scratch_types=[
            pltpu.VMEM_SHARED((N, D), jnp.float32),
            pltpu.VMEM((NIB, 2, EB), jnp.int32),
            pltpu.VMEM((NBUF, EB, D), jnp.float32),
            pltpu.VMEM((NBUF, EB, D), jnp.float32),
            pltpu.SemaphoreType.DMA((NIB,)),
            pltpu.SemaphoreType.DMA((NBUF,)),
            pltpu.SemaphoreType.DMA((NBUF,)),
            pltpu.SemaphoreType.DMA((NBUF,)),
        ],
    )(hin, eidx, ea)


def _tc_pre_body(h_ref, vn_ref, b_ref, hin_ref, seg_ref):
    i = pl.program_id(0)
    bk = b_ref[0]
    onehot = (bk.T == lax.broadcasted_iota(jnp.int32, (RB, G), 1)
              ).astype(jnp.float32)
    hin = h_ref[...] + jnp.dot(onehot, vn_ref[...],
                               preferred_element_type=jnp.float32)
    hin_ref[...] = hin
    contrib = jnp.dot(onehot.T, hin, preferred_element_type=jnp.float32)

    @pl.when(i == 0)
    def _():
        seg_ref[...] = contrib

    @pl.when(i != 0)
    def _():
        seg_ref[...] += contrib


@jax.jit
def _tc_pre(h, vn, batch3):
    return pl.pallas_call(
        _tc_pre_body,
        grid=(NBLK,),
        in_specs=[
            pl.BlockSpec((RB, D), lambda i: (i, 0)),
            pl.BlockSpec((G, D), lambda i: (0, 0)),
            pl.BlockSpec((1, 1, RB), lambda i: (i, 0, 0)),
        ],
        out_specs=[
            pl.BlockSpec((RB, D), lambda i: (i, 0)),
            pl.BlockSpec((G, D), lambda i: (0, 0)),
        ],
        out_shape=[
            jax.ShapeDtypeStruct((N, D), jnp.float32),
            jax.ShapeDtypeStruct((G, D), jnp.float32),
        ],
    )(h, vn, batch3)


def _tc_fused_body(hin_ref, agg_ref, seg_ref, vn_ref, b_ref, eps_ref, w1_ref,
                   a1_ref, c1_ref, w2_ref, a2_ref, c2_ref, vw1_ref, va1_ref,
                   vc1_ref, vw2_ref, va2_ref, vc2_ref,
                   hin_out, seg_out, vn_out, vn_scr):
    i = pl.program_id(0)

    @pl.when(i == 0)
    def _():
        vt = seg_ref[...] + vn_ref[...]
        v = jnp.dot(vt, vw1_ref[...].T, preferred_element_type=jnp.float32)
        v = v * va1_ref[...] + vc1_ref[...]
        v = jnp.maximum(v, 0.0)
        v = jnp.dot(v, vw2_ref[...].T, preferred_element_type=jnp.float32)
        v = v * va2_ref[...] + vc2_ref[...]
        v = jnp.maximum(v, 0.0)
        vn_scr[...] = v
        vn_out[...] = v

    z = (1.0 + eps_ref[0, 0]) * hin_ref[...] + agg_ref[0] + agg_ref[1]
    z1 = jnp.dot(z, w1_ref[...].T, preferred_element_type=jnp.float32)
    z1 = z1 * a1_ref[...] + c1_ref[...]
    z1 = jnp.maximum(z1, 0.0)
    z2 = jnp.dot(z1, w2_ref[...].T, preferred_element_type=jnp.float32)
    z2 = z2 * a2_ref[...] + c2_ref[...]
    h_new = jnp.maximum(z2, 0.0)

    bk = b_ref[0]
    onehot = (bk.T == lax.broadcasted_iota(jnp.int32, (RB, G), 1)
              ).astype(jnp.float32)
    hin_next = h_new + jnp.dot(onehot, vn_scr[...],
                               preferred_element_type=jnp.float32)
    hin_out[...] = hin_next
    contrib = jnp.dot(onehot.T, hin_next, preferred_element_type=jnp.float32)

    @pl.when(i == 0)
    def _():
        seg_out[...] = contrib

    @pl.when(i != 0)
    def _():
        seg_out[...] += contrib


@jax.jit
def _tc_fused(hin, agg2, seg, vn, batch3, eps, w1, a1, c1, w2, a2, c2,
              vw1, va1, vc1, vw2, va2, vc2):
    full = lambda i: (0, 0)
    return pl.pallas_call(
        _tc_fused_body,
        grid=(NBLK,),
        in_specs=[
            pl.BlockSpec((RB, D), lambda i: (i, 0)),
            pl.BlockSpec((NC, RB, D), lambda i: (0, i, 0)),
            pl.BlockSpec((G, D), full),
            pl.BlockSpec((G, D), full),
            pl.BlockSpec((1, 1, RB), lambda i: (i, 0, 0)),
            pl.BlockSpec((1, 1), full),
            pl.BlockSpec((2 * D, D), full),
            pl.BlockSpec((1, 2 * D), full),
            pl.BlockSpec((1, 2 * D), full),
            pl.BlockSpec((D, 2 * D), full),
            pl.BlockSpec((1, D), full),
            pl.BlockSpec((1, D), full),
            pl.BlockSpec((2 * D, D), full),
            pl.BlockSpec((1, 2 * D), full),
            pl.BlockSpec((1, 2 * D), full),
            pl.BlockSpec((D, 2 * D), full),
            pl.BlockSpec((1, D), full),
            pl.BlockSpec((1, D), full),
        ],
        out_specs=[
            pl.BlockSpec((RB, D), lambda i: (i, 0)),
            pl.BlockSpec((G, D), full),
            pl.BlockSpec((G, D), full),
        ],
        out_shape=[
            jax.ShapeDtypeStruct((N, D), jnp.float32),
            jax.ShapeDtypeStruct((G, D), jnp.float32),
            jax.ShapeDtypeStruct((G, D), jnp.float32),
        ],
        scratch_shapes=[pltpu.VMEM((G, D), jnp.float32)],
    )(hin, agg2, seg, vn, batch3, eps, w1, a1, c1, w2, a2, c2,
      vw1, va1, vc1, vw2, va2, vc2)


def _tc_final_body(hin_ref, agg_ref, b_ref, eps_ref, w1_ref, a1_ref, c1_ref,
                   w2_ref, a2_ref, c2_ref, wh_ref, bh_ref, out_ref,
                   sum_ref, cnt_ref):
    i = pl.program_id(0)
    z = (1.0 + eps_ref[0, 0]) * hin_ref[...] + agg_ref[0] + agg_ref[1]
    z1 = jnp.dot(z, w1_ref[...].T, preferred_element_type=jnp.float32)
    z1 = z1 * a1_ref[...] + c1_ref[...]
    z1 = jnp.maximum(z1, 0.0)
    z2 = jnp.dot(z1, w2_ref[...].T, preferred_element_type=jnp.float32)
    h_new = z2 * a2_ref[...] + c2_ref[...]

    bk = b_ref[0]
    onehot = (bk.T == lax.broadcasted_iota(jnp.int32, (RB, G), 1)
              ).astype(jnp.float32)

    @pl.when(i == 0)
    def _():
        sum_ref[...] = jnp.zeros((G, D), jnp.float32)
        cnt_ref[...] = jnp.zeros((G, D), jnp.float32)

    sum_ref[...] += jnp.dot(onehot.T, h_new,
                            preferred_element_type=jnp.float32)
    cnt_ref[...] += jnp.dot(onehot.T, jnp.ones((RB, D), jnp.float32),
                            preferred_element_type=jnp.float32)

    @pl.when(i == NBLK - 1)
    def _():
        hg = sum_ref[...] / jnp.maximum(cnt_ref[...], 1.0)
        out_ref[...] = jnp.dot(hg, wh_ref[...].T,
                               preferred_element_type=jnp.float32) + bh_ref[...]


@jax.jit
def _tc_final(hin, agg2, batch3, eps, w1, a1, c1, w2, a2, c2, wh_pad, bh_pad):
    full = lambda i: (0, 0)
    return pl.pallas_call(
        _tc_final_body,
        grid=(NBLK,),
        in_specs=[
            pl.BlockSpec((RB, D), lambda i: (i, 0)),
            pl.BlockSpec((NC, RB, D), lambda i: (0, i, 0)),
            pl.BlockSpec((1, 1, RB), lambda i: (i, 0, 0)),
            pl.BlockSpec((1, 1), full),
            pl.BlockSpec((2 * D, D), full),
            pl.BlockSpec((1, 2 * D), full),
            pl.BlockSpec((1, 2 * D), full),
            pl.BlockSpec((D, 2 * D), full),
            pl.BlockSpec((1, D), full),
            pl.BlockSpec((1, D), full),
            pl.BlockSpec((D, D), full),
            pl.BlockSpec((1, D), full),
        ],
        out_specs=pl.BlockSpec((G, D), full),
        out_shape=jax.ShapeDtypeStruct((G, D), jnp.float32),
        scratch_shapes=[
            pltpu.VMEM((G, D), jnp.float32),
            pltpu.VMEM((G, D), jnp.float32),
        ],
    )(hin, agg2, batch3, eps, w1, a1, c1, w2, a2, c2, wh_pad, bh_pad)


def kernel(x, edge_index, edge_attr, batch, params):
    eidx = jnp.stack([edge_index[0].reshape(NW, NCHUNK, EB),
                      edge_index[1].reshape(NW, NCHUNK, EB)], axis=2)
    batch3 = batch.reshape(NBLK, 1, RB)

    p = params
    wh_pad = jnp.zeros((D, D), jnp.float32).at[:C, :].set(p['Wh'])
    bh_pad = jnp.zeros((1, D), jnp.float32).at[0, :C].set(p['bh'])

    def mparams(l):
        return (
            p['eps'][l].reshape(1, 1),
            p['W1'][l], p['bn1_g'][l].reshape(1, 2 * D),
            (p['b1'][l] * p['bn1_g'][l] + p['bn1_b'][l]).reshape(1, 2 * D),
            p['W2'][l], p['bn_g'][l].reshape(1, D),
            (p['b2'][l] * p['bn_g'][l] + p['bn_b'][l]).reshape(1, D),
        )

    def vparams(l):
        return (
            p['vW1'][l], p['vbn1_g'][l].reshape(1, 2 * D),
            (p['vb1'][l] * p['vbn1_g'][l] + p['vbn1_b'][l]).reshape(1, 2 * D),
            p['vW2'][l], p['vbn2_g'][l].reshape(1, D),
            (p['vb2'][l] * p['vbn2_g'][l] + p['vbn2_b'][l]).reshape(1, D),
        )

    vn = jnp.zeros((G, D), jnp.float32)
    hin, seg = _tc_pre(x, vn, batch3)
    for l in range(L - 1):
        agg2 = _sc_message(hin, eidx, edge_attr)
        hin, seg, vn = _tc_fused(hin, agg2, seg, vn, batch3,
                                 *mparams(l), *vparams(l))
    agg2 = _sc_message(hin, eidx, edge_attr)
    out = _tc_final(hin, agg2, batch3, *mparams(L - 1), wh_pad, bh_pad)
    return out[:, :C]
